# baseline (speedup 1.0000x reference)
import functools

import jax
import jax.numpy as jnp
from jax import lax
from jax.experimental import pallas as pl
from jax.experimental.pallas import tpu as pltpu
from jax.experimental.pallas import tpu_sc as plsc

NC = 2
NS = 16
QC = 64
NQ = 4


def _invsqrt(x):
    i = lax.bitcast_convert_type(x, jnp.int32)
    y = lax.bitcast_convert_type(jnp.int32(0x5F3759DF) - (i >> 1), jnp.float32)
    for _ in range(3):
        y = y * (1.5 - 0.5 * x * y * y)
    return y


def _take(v, t):
    return jnp.take_along_axis(v, jnp.full((16,), t, jnp.int32), axis=0)


def _sc_deg_body(npad, erows, rpt,
                 dst2, ew2, edeg,
                 deg_sp, dstbuf, ewbuf, zdeg, ssem):
    c = lax.axis_index("c")
    s = lax.axis_index("s")
    rows_t = npad // NS
    t0 = s * rows_t
    erow0 = s * rpt
    KA = rpt // 8

    zv = jnp.zeros((16,), jnp.float32)
    @plsc.parallel_loop(0, rows_t // 16, unroll=8)
    def _(i):
        zdeg[pl.ds(i * 16, 16)] = zv
    pltpu.sync_copy(zdeg, deg_sp.at[pl.ds(t0, rows_t)])
    plsc.subcore_barrier()

    def load_deg_idx(k, par):
        r0 = erow0 + k * 8
        pltpu.sync_copy(dst2.at[pl.ds(r0, 8)], dstbuf.at[pl.ds(8 * par, 8)])
        pltpu.sync_copy(ew2.at[pl.ds(r0, 8)], ewbuf.at[pl.ds(8 * par, 8)])

    def start_deg(par):
        for j in range(8):
            pltpu.async_copy(ewbuf.at[8 * par + j],
                             deg_sp.at[dstbuf.at[8 * par + j]],
                             ssem.at[par], add=True)

    def wait_deg(par):
        pltpu.make_async_copy(ew2.at[pl.ds(0, 8)],
                              ewbuf.at[pl.ds(8 * par, 8)],
                              ssem.at[par]).wait()

    with jax.named_scope("sc_deg"):
        load_deg_idx(0, 0)
        def dega(k, _):
            par = lax.rem(k, 2)
            pnx = 1 - par
            @pl.when(k + 1 < KA)
            def _():
                @pl.when(k >= 1)
                def _():
                    wait_deg(pnx)
                load_deg_idx(k + 1, pnx)
            start_deg(par)
            return 0
        lax.fori_loop(0, KA, dega, 0)
        wait_deg((KA - 1) % 2)
        if KA >= 2:
            wait_deg((KA - 2) % 2)
        plsc.subcore_barrier()
    @pl.when(c == 0)
    def _():
        pltpu.sync_copy(deg_sp.at[pl.ds(t0, rows_t)],
                        edeg.at[pl.ds(t0, rows_t)])


def _sc_body(npad, nrows, erows, rpt,
             xq, src2, dst2, ew2, edeg, agg,
             acc_sp, xsp, dinv_v, rowsbuf, srcbuf, dstbuf, ewbuf,
             normbuf, gsem, ssem, isem):
    c = lax.axis_index("c")
    s = lax.axis_index("s")
    rows_t = npad // NS
    t0 = s * rows_t
    erow0 = s * rpt
    K = rpt

    with jax.named_scope("sc_dinv"):
        pltpu.sync_copy(edeg, dinv_v)
        @plsc.parallel_loop(0, npad // 16, unroll=4)
        def _(i):
            d = dinv_v[pl.ds(i * 16, 16)] + 1.0
            dinv_v[pl.ds(i * 16, 16)] = jnp.where(d > 0, _invsqrt(d), 0.0)

    def load_super_idx(sc2, sp):
        r0 = erow0 + 8 * sc2
        pltpu.sync_copy(src2.at[pl.ds(r0, 8)], srcbuf.at[pl.ds(8 * sp, 8)])
        pltpu.sync_copy(dst2.at[pl.ds(r0, 8)], dstbuf.at[pl.ds(8 * sp, 8)])
        pltpu.sync_copy(ew2.at[pl.ds(r0, 8)], ewbuf.at[pl.ds(8 * sp, 8)])

    def start_super_idx(sc2, sp):
        r0 = erow0 + 8 * sc2
        pltpu.async_copy(src2.at[pl.ds(r0, 8)], srcbuf.at[pl.ds(8 * sp, 8)],
                         isem)
        pltpu.async_copy(dst2.at[pl.ds(r0, 8)], dstbuf.at[pl.ds(8 * sp, 8)],
                         isem)
        pltpu.async_copy(ew2.at[pl.ds(r0, 8)], ewbuf.at[pl.ds(8 * sp, 8)],
                         isem)

    def wait_super_idx(sp):
        pltpu.make_async_copy(src2.at[pl.ds(0, 8)],
                              srcbuf.at[pl.ds(8 * sp, 8)], isem).wait()
        pltpu.make_async_copy(dst2.at[pl.ds(0, 8)],
                              dstbuf.at[pl.ds(8 * sp, 8)], isem).wait()
        pltpu.make_async_copy(ew2.at[pl.ds(0, 8)],
                              ewbuf.at[pl.ds(8 * sp, 8)], isem).wait()

    def start_gather(g, par, irow):
        pltpu.async_copy(xsp.at[srcbuf.at[irow]],
                         rowsbuf.at[pl.ds(128 * par, 128)],
                         gsem.at[par])

    def wait_gather(g, par):
        pltpu.make_async_copy(xq.at[pl.ds(0, 128), pl.ds(0, QC)],
                              rowsbuf.at[pl.ds(128 * par, 128)],
                              gsem.at[par]).wait()

    def start_scatter(par, irow):
        pltpu.async_copy(rowsbuf.at[pl.ds(128 * par, 128)],
                         acc_sp.at[dstbuf.at[irow]],
                         ssem.at[par], add=True)

    def wait_scatter(g, par):
        pltpu.make_async_copy(xq.at[pl.ds(0, 128), pl.ds(0, QC)],
                              rowsbuf.at[pl.ds(128 * par, 128)],
                              ssem.at[par]).wait()

    def norms(par, irow):
        for u in range(8):
            sl = pl.ds(u * 16, 16)
            nrm = (plsc.load_gather(dinv_v, [srcbuf[irow, sl]])
                   * ewbuf[irow, sl]
                   * plsc.load_gather(dinv_v, [dstbuf[irow, sl]]))
            normbuf[pl.ds(128 * par + u * 16, 16)] = nrm

    def scale(par):
        base = 128 * par
        @plsc.parallel_loop(0, 8, unroll=2)
        def _(gi):
            nv = normbuf[pl.ds(base + 16 * gi, 16)]
            for t in range(16):
                nsp = _take(nv, t)
                r = base + 16 * gi + t
                for u in range(QC // 16):
                    sl = pl.ds(u * 16, 16)
                    rowsbuf[r, sl] = rowsbuf[r, sl] * nsp

    for p in range(2):
        g = 2 * p + c

        with jax.named_scope("sc_selfinit"):
            ts = nrows // rows_t
            tail = nrows % rows_t
            @pl.when(s < ts)
            def _():
                pltpu.sync_copy(xq.at[pl.ds(t0, rows_t), pl.ds(QC * g, QC)],
                                xsp.at[pl.ds(t0, rows_t)])
            if tail:
                @pl.when(s == ts)
                def _():
                    pltpu.sync_copy(
                        xq.at[pl.ds(ts * rows_t, tail), pl.ds(QC * g, QC)],
                        xsp.at[pl.ds(ts * rows_t, tail)])
            hh = rows_t // 2
            for h in range(2):
                pltpu.sync_copy(xsp.at[pl.ds(t0 + h * hh, hh)],
                                rowsbuf.at[pl.ds(0, hh)])
                @plsc.parallel_loop(0, hh // 16, unroll=2)
                def _(gi):
                    dv = dinv_v[pl.ds(t0 + h * hh + 16 * gi, 16)]
                    dsq = dv * dv
                    for t in range(16):
                        nsp = _take(dsq, t)
                        r = 16 * gi + t
                        for u in range(QC // 16):
                            sl = pl.ds(u * 16, 16)
                            rowsbuf[r, sl] = rowsbuf[r, sl] * nsp
                pltpu.sync_copy(rowsbuf.at[pl.ds(0, hh)],
                                acc_sp.at[pl.ds(t0 + h * hh, hh)])
            plsc.subcore_barrier()

        SB = K // 8
        with jax.named_scope("sc_edges"):
            load_super_idx(0, 0)
            start_gather(g, 0, 0)
            def edgec(sc2, _):
                sp = lax.rem(sc2, 2)
                spn = lax.rem(sc2 + 1, 2)
                @pl.when(sc2 >= 1)
                def _():
                    wait_scatter(g, lax.rem(8 * sc2 - 2, 3))
                    wait_scatter(g, lax.rem(8 * sc2 - 1, 3))
                @pl.when(sc2 + 1 < SB)
                def _():
                    start_super_idx(sc2 + 1, spn)
                for j in range(8):
                    m = 8 * sc2 + j
                    par = lax.rem(m, 3)
                    if j >= 2:
                        wait_scatter(g, lax.rem(m + 1, 3))
                    @pl.when(m + 1 < K)
                    def _():
                        if j == 7:
                            wait_super_idx(spn)
                        nirow = (8 * sp + j + 1) if j < 7 else 8 * spn
                        start_gather(g, lax.rem(m + 1, 3), nirow)
                    norms(par, 8 * sp + j)
                    wait_gather(g, par)
                    scale(par)
                    start_scatter(par, 8 * sp + j)
                return 0
            lax.fori_loop(0, SB, edgec, 0)
            wait_scatter(g, (K - 2) % 3)
            wait_scatter(g, (K - 1) % 3)
            plsc.subcore_barrier()

        with jax.named_scope("sc_out"):
            pltpu.sync_copy(acc_sp.at[pl.ds(t0, rows_t)],
                            agg.at[g].at[pl.ds(t0, rows_t)])


def _sc_aggregate(xq, src2, dst2, ew2, npad, nrows, erows):
    rpt = erows // NS
    mesh = plsc.VectorSubcoreMesh(core_axis_name="c", subcore_axis_name="s",
                                  num_cores=NC, num_subcores=NS)
    cp = pltpu.CompilerParams(needs_layout_passes=False,
                              use_tc_tiling_on_sc=False)
    edeg = pl.kernel(
        functools.partial(_sc_deg_body, npad, erows, rpt),
        out_type=jax.ShapeDtypeStruct((npad,), jnp.float32),
        mesh=mesh,
        compiler_params=cp,
        scratch_types=[
            pltpu.VMEM_SHARED((npad,), jnp.float32),
            pltpu.VMEM((16, 128), jnp.int32),
            pltpu.VMEM((16, 128), jnp.float32),
            pltpu.VMEM((npad // NS,), jnp.float32),
            pltpu.SemaphoreType.DMA((2,)),
        ],
    )(dst2, ew2)
    return pl.kernel(
        functools.partial(_sc_body, npad, nrows, erows, rpt),
        out_type=jax.ShapeDtypeStruct((NQ, npad, QC), jnp.float32),
        mesh=mesh,
        compiler_params=cp,
        scratch_types=[
            pltpu.VMEM_SHARED((npad, QC), jnp.float32),
            pltpu.VMEM_SHARED((npad, QC), jnp.float32),
            pltpu.VMEM((npad,), jnp.float32),
            pltpu.VMEM((max(384, npad // NS // 2), QC), jnp.float32),
            pltpu.VMEM((16, 128), jnp.int32),
            pltpu.VMEM((16, 128), jnp.int32),
            pltpu.VMEM((16, 128), jnp.float32),
            pltpu.VMEM((384,), jnp.float32),
            pltpu.SemaphoreType.DMA((3,)),
            pltpu.SemaphoreType.DMA((3,)),
            pltpu.SemaphoreType.DMA,
        ],
    )(xq, src2, dst2, ew2, edeg)


def _dense_body(yq, h, Wz, bz, Wr, br, Wh, bh,
                Lz1, Lz2, Lzb, Lr1, Lr2, Lrb, Lh1, Lh2, Lhb,
                enc_w, enc_b, node_w, node_b, out_w, out_b,
                pred, h0n):
    f32 = jnp.float32
    dot = lambda a, b: lax.dot_general(a, b, (((1,), (0,)), ((), ())),
                                       preferred_element_type=f32)
    y = jnp.concatenate([yq[0], yq[1], yq[2], yq[3]], axis=1)
    H = h[...]
    cz = dot(y, Wz[...]) + bz[...]
    cr = dot(y, Wr[...]) + br[...]
    ch = dot(y, Wh[...]) + bh[...]
    Z = jax.nn.sigmoid(dot(cz, Lz1[...]) + dot(H, Lz2[...]) + Lzb[...])
    R = jax.nn.sigmoid(dot(cr, Lr1[...]) + dot(H, Lr2[...]) + Lrb[...])
    Ht = jnp.tanh(dot(ch, Lh1[...]) + dot(H * R, Lh2[...]) + Lhb[...])
    h0_new = Z * H + (1.0 - Z) * Ht
    hr = jax.nn.relu(h0_new)
    z = dot(hr, enc_w[...]) + enc_b[...]
    hh = jax.nn.relu(dot(z, node_w[...]) + node_b[...])
    pred[...] = dot(hh, out_w[...]) + out_b[...]
    h0n[...] = h0_new


def kernel(node_feat, src, dst, edge_weight, node_ids, h_0,
           Wz, bz, Wr, br, Wh, bh,
           Lz_w, Lz_b, Lr_w, Lr_b, Lh_w, Lh_b,
           enc_w, enc_b, node_w, node_b, out_w, out_b):
    N, D = node_feat.shape
    HD = h_0.shape[1]
    C = out_w.shape[1]
    E = src.shape[0]

    npad = -(-N // (NS * 128)) * (NS * 128)
    erows = -(-E // (128 * NS * 8)) * (NS * 8)
    epad = erows * 128

    pad = epad - E
    src2 = jnp.concatenate(
        [src, jnp.zeros((pad,), src.dtype)]).reshape(erows, 128)
    dst2 = jnp.concatenate(
        [dst, jnp.full((pad,), npad - 1, dst.dtype)]).reshape(erows, 128)
    ew2 = jnp.concatenate(
        [edge_weight, jnp.zeros((pad,), edge_weight.dtype)]).reshape(erows, 128)

    yq = _sc_aggregate(node_feat, src2, dst2, ew2, npad, N, erows)

    BLK = 1000
    grid = N // BLK
    Lz1, Lz2 = Lz_w[:HD], Lz_w[HD:]
    Lr1, Lr2 = Lr_w[:HD], Lr_w[HD:]
    Lh1, Lh2 = Lh_w[:HD], Lh_w[HD:]
    row2 = lambda v: v.reshape(1, -1)

    full = lambda a: pl.BlockSpec(a.shape, lambda i: (0,) * a.ndim)
    weights = [Wz, row2(bz), Wr, row2(br), Wh, row2(bh),
               Lz1, Lz2, row2(Lz_b), Lr1, Lr2, row2(Lr_b),
               Lh1, Lh2, row2(Lh_b),
               enc_w, row2(enc_b), node_w, row2(node_b),
               out_w, row2(out_b)]
    pred, h0_new = pl.pallas_call(
        _dense_body,
        grid=(grid,),
        in_specs=[pl.BlockSpec((NQ, BLK, QC), lambda i: (0, i, 0)),
                  pl.BlockSpec((BLK, HD), lambda i: (i, 0))]
                 + [full(w) for w in weights],
        out_specs=[pl.BlockSpec((BLK, C), lambda i: (i, 0)),
                   pl.BlockSpec((BLK, HD), lambda i: (i, 0))],
        out_shape=[jax.ShapeDtypeStruct((N, C), jnp.float32),
                   jax.ShapeDtypeStruct((N, HD), jnp.float32)],
    )(yq, h_0, *weights)

    return (pred, h0_new)


# speedup vs baseline: 1.0216x; 1.0216x over previous
import functools

import jax
import jax.numpy as jnp
from jax import lax
from jax.experimental import pallas as pl
from jax.experimental.pallas import tpu as pltpu
from jax.experimental.pallas import tpu_sc as plsc

NC = 2
NS = 16
QC = 64
NQ = 4


def _invsqrt(x):
    i = lax.bitcast_convert_type(x, jnp.int32)
    y = lax.bitcast_convert_type(jnp.int32(0x5F3759DF) - (i >> 1), jnp.float32)
    for _ in range(3):
        y = y * (1.5 - 0.5 * x * y * y)
    return y


def _take(v, t):
    return jnp.take_along_axis(v, jnp.full((16,), t, jnp.int32), axis=0)


def _sc_body(npad, nrows, erows, rpt,
             xq, src2, dst2, ew2, agg,
             acc_sp, xsp, deg_sp, dinv_v, rowsbuf, srcbuf, dstbuf, ewbuf,
             normbuf, zdeg, gsem, ssem, isem):
    c = lax.axis_index("c")
    s = lax.axis_index("s")
    rows_t = npad // NS
    t0 = s * rows_t
    erow0 = s * rpt
    K = rpt
    KA = rpt // 8

    zv = jnp.zeros((16,), jnp.float32)

    @plsc.parallel_loop(0, rows_t // 16, unroll=8)
    def _(i):
        zdeg[pl.ds(i * 16, 16)] = zv
    pltpu.sync_copy(zdeg, deg_sp.at[pl.ds(t0, rows_t)])
    plsc.subcore_barrier()

    def load_deg_idx(k, par):
        r0 = erow0 + k * 8
        pltpu.sync_copy(dst2.at[pl.ds(r0, 8)], dstbuf.at[pl.ds(8 * par, 8)])
        pltpu.sync_copy(ew2.at[pl.ds(r0, 8)], ewbuf.at[pl.ds(8 * par, 8)])

    def start_deg(par):
        for j in range(8):
            pltpu.async_copy(ewbuf.at[8 * par + j],
                             deg_sp.at[dstbuf.at[8 * par + j]],
                             ssem.at[par], add=True)

    def wait_deg(par):
        pltpu.make_async_copy(ew2.at[pl.ds(0, 8)],
                              ewbuf.at[pl.ds(8 * par, 8)],
                              ssem.at[par]).wait()

    with jax.named_scope("sc_deg"):
        load_deg_idx(0, 0)
        def dega(k, _):
            par = lax.rem(k, 2)
            pnx = 1 - par
            @pl.when(k + 1 < KA)
            def _():
                @pl.when(k >= 1)
                def _():
                    wait_deg(pnx)
                load_deg_idx(k + 1, pnx)
            start_deg(par)
            return 0
        lax.fori_loop(0, KA, dega, 0)
        wait_deg((KA - 1) % 2)
        if KA >= 2:
            wait_deg((KA - 2) % 2)
        plsc.subcore_barrier()

    with jax.named_scope("sc_dinv"):
        pltpu.sync_copy(deg_sp, dinv_v)
        @plsc.parallel_loop(0, npad // 16, unroll=4)
        def _(i):
            d = dinv_v[pl.ds(i * 16, 16)] + 1.0
            dinv_v[pl.ds(i * 16, 16)] = jnp.where(d > 0, _invsqrt(d), 0.0)

    def load_super_idx(sc2, sp):
        r0 = erow0 + 8 * sc2
        pltpu.sync_copy(src2.at[pl.ds(r0, 8)], srcbuf.at[pl.ds(8 * sp, 8)])
        pltpu.sync_copy(dst2.at[pl.ds(r0, 8)], dstbuf.at[pl.ds(8 * sp, 8)])
        pltpu.sync_copy(ew2.at[pl.ds(r0, 8)], ewbuf.at[pl.ds(8 * sp, 8)])

    def start_super_idx(sc2, sp):
        r0 = erow0 + 8 * sc2
        pltpu.async_copy(src2.at[pl.ds(r0, 8)], srcbuf.at[pl.ds(8 * sp, 8)],
                         isem)
        pltpu.async_copy(dst2.at[pl.ds(r0, 8)], dstbuf.at[pl.ds(8 * sp, 8)],
                         isem)
        pltpu.async_copy(ew2.at[pl.ds(r0, 8)], ewbuf.at[pl.ds(8 * sp, 8)],
                         isem)

    def wait_super_idx(sp):
        pltpu.make_async_copy(src2.at[pl.ds(0, 8)],
                              srcbuf.at[pl.ds(8 * sp, 8)], isem).wait()
        pltpu.make_async_copy(dst2.at[pl.ds(0, 8)],
                              dstbuf.at[pl.ds(8 * sp, 8)], isem).wait()
        pltpu.make_async_copy(ew2.at[pl.ds(0, 8)],
                              ewbuf.at[pl.ds(8 * sp, 8)], isem).wait()

    def start_gather(g, par, irow):
        pltpu.async_copy(xsp.at[srcbuf.at[irow]],
                         rowsbuf.at[pl.ds(128 * par, 128)],
                         gsem.at[par])

    def wait_gather(g, par):
        pltpu.make_async_copy(xq.at[pl.ds(0, 128), pl.ds(0, QC)],
                              rowsbuf.at[pl.ds(128 * par, 128)],
                              gsem.at[par]).wait()

    def start_scatter(par, irow):
        pltpu.async_copy(rowsbuf.at[pl.ds(128 * par, 128)],
                         acc_sp.at[dstbuf.at[irow]],
                         ssem.at[par], add=True)

    def wait_scatter(g, par):
        pltpu.make_async_copy(xq.at[pl.ds(0, 128), pl.ds(0, QC)],
                              rowsbuf.at[pl.ds(128 * par, 128)],
                              ssem.at[par]).wait()

    def norms(par, irow):
        for u in range(8):
            sl = pl.ds(u * 16, 16)
            nrm = (plsc.load_gather(dinv_v, [srcbuf[irow, sl]])
                   * ewbuf[irow, sl]
                   * plsc.load_gather(dinv_v, [dstbuf[irow, sl]]))
            normbuf[pl.ds(128 * par + u * 16, 16)] = nrm

    def scale(par):
        base = 128 * par
        @plsc.parallel_loop(0, 8, unroll=2)
        def _(gi):
            nv = normbuf[pl.ds(base + 16 * gi, 16)]
            for t in range(16):
                nsp = _take(nv, t)
                r = base + 16 * gi + t
                for u in range(QC // 16):
                    sl = pl.ds(u * 16, 16)
                    rowsbuf[r, sl] = rowsbuf[r, sl] * nsp

    for p in range(2):
        g = 2 * p + c

        with jax.named_scope("sc_selfinit"):
            ts = nrows // rows_t
            tail = nrows % rows_t
            @pl.when(s < ts)
            def _():
                pltpu.sync_copy(xq.at[pl.ds(t0, rows_t), pl.ds(QC * g, QC)],
                                xsp.at[pl.ds(t0, rows_t)])
            if tail:
                @pl.when(s == ts)
                def _():
                    pltpu.sync_copy(
                        xq.at[pl.ds(ts * rows_t, tail), pl.ds(QC * g, QC)],
                        xsp.at[pl.ds(ts * rows_t, tail)])
            hh = rows_t // 2
            for h in range(2):
                pltpu.sync_copy(xsp.at[pl.ds(t0 + h * hh, hh)],
                                rowsbuf.at[pl.ds(0, hh)])
                @plsc.parallel_loop(0, hh // 16, unroll=2)
                def _(gi):
                    dv = dinv_v[pl.ds(t0 + h * hh + 16 * gi, 16)]
                    dsq = dv * dv
                    for t in range(16):
                        nsp = _take(dsq, t)
                        r = 16 * gi + t
                        for u in range(QC // 16):
                            sl = pl.ds(u * 16, 16)
                            rowsbuf[r, sl] = rowsbuf[r, sl] * nsp
                pltpu.sync_copy(rowsbuf.at[pl.ds(0, hh)],
                                acc_sp.at[pl.ds(t0 + h * hh, hh)])
            plsc.subcore_barrier()

        SB = K // 8
        with jax.named_scope("sc_edges"):
            load_super_idx(0, 0)
            start_gather(g, 0, 0)
            def edgec(sc2, _):
                sp = lax.rem(sc2, 2)
                spn = lax.rem(sc2 + 1, 2)
                @pl.when(sc2 >= 1)
                def _():
                    wait_scatter(g, lax.rem(8 * sc2 - 2, 3))
                    wait_scatter(g, lax.rem(8 * sc2 - 1, 3))
                @pl.when(sc2 + 1 < SB)
                def _():
                    start_super_idx(sc2 + 1, spn)
                for j in range(8):
                    m = 8 * sc2 + j
                    par = lax.rem(m, 3)
                    if j >= 2:
                        wait_scatter(g, lax.rem(m + 1, 3))
                    @pl.when(m + 1 < K)
                    def _():
                        if j == 7:
                            wait_super_idx(spn)
                        nirow = (8 * sp + j + 1) if j < 7 else 8 * spn
                        start_gather(g, lax.rem(m + 1, 3), nirow)
                    norms(par, 8 * sp + j)
                    wait_gather(g, par)
                    scale(par)
                    start_scatter(par, 8 * sp + j)
                return 0
            lax.fori_loop(0, SB, edgec, 0)
            wait_scatter(g, (K - 2) % 3)
            wait_scatter(g, (K - 1) % 3)
            plsc.subcore_barrier()

        with jax.named_scope("sc_out"):
            pltpu.sync_copy(acc_sp.at[pl.ds(t0, rows_t)],
                            agg.at[g].at[pl.ds(t0, rows_t)])


def _sc_aggregate(xq, src2, dst2, ew2, npad, nrows, erows):
    rpt = erows // NS
    mesh = plsc.VectorSubcoreMesh(core_axis_name="c", subcore_axis_name="s",
                                  num_cores=NC, num_subcores=NS)
    return pl.kernel(
        functools.partial(_sc_body, npad, nrows, erows, rpt),
        out_type=jax.ShapeDtypeStruct((NQ, npad, QC), jnp.float32),
        mesh=mesh,
        compiler_params=pltpu.CompilerParams(needs_layout_passes=False,
                                             use_tc_tiling_on_sc=False),
        scratch_types=[
            pltpu.VMEM_SHARED((npad, QC), jnp.float32),
            pltpu.VMEM_SHARED((npad, QC), jnp.float32),
            pltpu.VMEM_SHARED((npad,), jnp.float32),
            pltpu.VMEM((npad,), jnp.float32),
            pltpu.VMEM((max(384, npad // NS // 2), QC), jnp.float32),
            pltpu.VMEM((16, 128), jnp.int32),
            pltpu.VMEM((16, 128), jnp.int32),
            pltpu.VMEM((16, 128), jnp.float32),
            pltpu.VMEM((384,), jnp.float32),
            pltpu.VMEM((npad // NS,), jnp.float32),
            pltpu.SemaphoreType.DMA((3,)),
            pltpu.SemaphoreType.DMA((3,)),
            pltpu.SemaphoreType.DMA,
        ],
    )(xq, src2, dst2, ew2)


def _dense_body(yq, h, Wz, bz, Wr, br, Wh, bh,
                Lz1, Lz2, Lzb, Lr1, Lr2, Lrb, Lh1, Lh2, Lhb,
                enc_w, enc_b, node_w, node_b, out_w, out_b,
                pred, h0n):
    f32 = jnp.float32
    dot = lambda a, b: lax.dot_general(a, b, (((1,), (0,)), ((), ())),
                                       preferred_element_type=f32)
    y = jnp.concatenate([yq[0], yq[1], yq[2], yq[3]], axis=1)
    H = h[...]
    cz = dot(y, Wz[...]) + bz[...]
    cr = dot(y, Wr[...]) + br[...]
    ch = dot(y, Wh[...]) + bh[...]
    Z = jax.nn.sigmoid(dot(cz, Lz1[...]) + dot(H, Lz2[...]) + Lzb[...])
    R = jax.nn.sigmoid(dot(cr, Lr1[...]) + dot(H, Lr2[...]) + Lrb[...])
    Ht = jnp.tanh(dot(ch, Lh1[...]) + dot(H * R, Lh2[...]) + Lhb[...])
    h0_new = Z * H + (1.0 - Z) * Ht
    hr = jax.nn.relu(h0_new)
    z = dot(hr, enc_w[...]) + enc_b[...]
    hh = jax.nn.relu(dot(z, node_w[...]) + node_b[...])
    pred[...] = dot(hh, out_w[...]) + out_b[...]
    h0n[...] = h0_new


def kernel(node_feat, src, dst, edge_weight, node_ids, h_0,
           Wz, bz, Wr, br, Wh, bh,
           Lz_w, Lz_b, Lr_w, Lr_b, Lh_w, Lh_b,
           enc_w, enc_b, node_w, node_b, out_w, out_b):
    N, D = node_feat.shape
    HD = h_0.shape[1]
    C = out_w.shape[1]
    E = src.shape[0]

    npad = -(-N // (NS * 128)) * (NS * 128)
    erows = -(-E // (128 * NS * 8)) * (NS * 8)
    epad = erows * 128

    pad = epad - E
    src2 = jnp.concatenate(
        [src, jnp.zeros((pad,), src.dtype)]).reshape(erows, 128)
    dst2 = jnp.concatenate(
        [dst, jnp.full((pad,), npad - 1, dst.dtype)]).reshape(erows, 128)
    ew2 = jnp.concatenate(
        [edge_weight, jnp.zeros((pad,), edge_weight.dtype)]).reshape(erows, 128)

    yq = _sc_aggregate(node_feat, src2, dst2, ew2, npad, N, erows)

    BLK = 1000
    grid = N // BLK
    Lz1, Lz2 = Lz_w[:HD], Lz_w[HD:]
    Lr1, Lr2 = Lr_w[:HD], Lr_w[HD:]
    Lh1, Lh2 = Lh_w[:HD], Lh_w[HD:]
    row2 = lambda v: v.reshape(1, -1)

    full = lambda a: pl.BlockSpec(a.shape, lambda i: (0,) * a.ndim)
    weights = [Wz, row2(bz), Wr, row2(br), Wh, row2(bh),
               Lz1, Lz2, row2(Lz_b), Lr1, Lr2, row2(Lr_b),
               Lh1, Lh2, row2(Lh_b),
               enc_w, row2(enc_b), node_w, row2(node_b),
               out_w, row2(out_b)]
    pred, h0_new = pl.pallas_call(
        _dense_body,
        grid=(grid,),
        in_specs=[pl.BlockSpec((NQ, BLK, QC), lambda i: (0, i, 0)),
                  pl.BlockSpec((BLK, HD), lambda i: (i, 0))]
                 + [full(w) for w in weights],
        out_specs=[pl.BlockSpec((BLK, C), lambda i: (i, 0)),
                   pl.BlockSpec((BLK, HD), lambda i: (i, 0))],
        out_shape=[jax.ShapeDtypeStruct((N, C), jnp.float32),
                   jax.ShapeDtypeStruct((N, HD), jnp.float32)],
    )(yq, h_0, *weights)

    return (pred, h0_new)
